# baseline (speedup 1.0000x reference)
import functools
import math

import jax
import jax.numpy as jnp
from jax import lax
from jax.experimental import pallas as pl
from jax.experimental.pallas import tpu as pltpu
from jax.experimental.pallas import tpu_sc as plsc

D_MODEL = 64
D_PAD = 128
S_REAL = 50
S_PAD = 64
SCALE = math.sqrt(D_MODEL)

NC = 2
NS = 16
NW = NC * NS
N_CHUNKS = 8
CHUNK = 512


def _make_sc_gather_chunk(k, n2):
    rows_per_chunk = n2 // N_CHUNKS
    b_per_w = rows_per_chunk // NW
    assert b_per_w % CHUNK == 0
    n_steps = b_per_w // CHUNK
    mesh = plsc.VectorSubcoreMesh(core_axis_name="c", subcore_axis_name="s")

    @functools.partial(
        pl.kernel,
        mesh=mesh,
        out_type=jax.ShapeDtypeStruct((rows_per_chunk, D_PAD), jnp.float32),
        scratch_types=[
            pltpu.VMEM((CHUNK,), jnp.int32),
            pltpu.VMEM((CHUNK, D_PAD), jnp.float32),
            pltpu.SemaphoreType.DMA,
        ],
    )
    def gather_kernel(table_hbm, idx_hbm, out_hbm, idx_v, rows_v, sem):
        wid = lax.axis_index("s") * NC + lax.axis_index("c")
        base = wid * b_per_w

        @pl.loop(0, n_steps)
        def _(c):
            off = base + c * CHUNK
            pltpu.sync_copy(idx_hbm.at[pl.ds(k * rows_per_chunk + off, CHUNK)], idx_v)
            pltpu.async_copy(table_hbm.at[idx_v], rows_v, sem).wait()
            pltpu.sync_copy(rows_v, out_hbm.at[pl.ds(off, CHUNK)])

    return gather_kernel


def kernel(x, weight):
    b, s = x.shape
    xi = x.astype(jnp.int32)
    idx56 = jnp.concatenate([xi, xi[:, : S_PAD - S_REAL]], axis=1)
    n2 = b * S_PAD
    idx_flat = idx56.reshape(n2)
    table = jnp.pad(weight, ((0, 0), (0, D_PAD - D_MODEL)))

    b_chunk = b // N_CHUNKS
    parts = []
    for k in range(N_CHUNKS):
        g = _make_sc_gather_chunk(k, n2)(table, idx_flat)
        g3 = g.reshape(b_chunk, S_PAD, D_PAD)
        parts.append(g3[:, :S_REAL, :D_MODEL] * SCALE)
    return jnp.concatenate(parts, axis=0)


# speedup vs baseline: 1.1055x; 1.1055x over previous
import functools
import math

import jax
import jax.numpy as jnp
from jax import lax
from jax.experimental import pallas as pl
from jax.experimental.pallas import tpu as pltpu
from jax.experimental.pallas import tpu_sc as plsc

D_MODEL = 64
D_PAD = 128
S_REAL = 50
S_PAD = 56
SCALE = math.sqrt(D_MODEL)

NC = 2
NS = 16
NW = NC * NS
N_CHUNKS = 8
CHUNK = 896


def _make_sc_gather_chunk(k, n2):
    rows_per_chunk = n2 // N_CHUNKS
    b_per_w = rows_per_chunk // NW
    assert b_per_w % CHUNK == 0
    n_steps = b_per_w // CHUNK
    mesh = plsc.VectorSubcoreMesh(core_axis_name="c", subcore_axis_name="s")

    @functools.partial(
        pl.kernel,
        mesh=mesh,
        out_type=jax.ShapeDtypeStruct((rows_per_chunk, D_PAD), jnp.float32),
        scratch_types=[
            pltpu.VMEM((CHUNK,), jnp.int32),
            pltpu.VMEM((CHUNK, D_PAD), jnp.float32),
            pltpu.SemaphoreType.DMA,
        ],
    )
    def gather_kernel(table_hbm, idx_hbm, out_hbm, idx_v, rows_v, sem):
        wid = lax.axis_index("s") * NC + lax.axis_index("c")
        base = wid * b_per_w

        @pl.loop(0, n_steps)
        def _(c):
            off = base + c * CHUNK
            pltpu.sync_copy(idx_hbm.at[pl.ds(k * rows_per_chunk + off, CHUNK)], idx_v)
            pltpu.async_copy(table_hbm.at[idx_v], rows_v, sem).wait()
            pltpu.sync_copy(rows_v, out_hbm.at[pl.ds(off, CHUNK)])

    return gather_kernel


def _tc_prep(wt):
    v = wt.shape[1]
    blk = 1024
    grid = (v + blk - 1) // blk

    def body(w_ref, o_ref):
        o_ref[:, :D_MODEL] = w_ref[...].T * SCALE

    return pl.pallas_call(
        body,
        out_shape=jax.ShapeDtypeStruct((v, D_PAD), jnp.float32),
        grid=(grid,),
        in_specs=[pl.BlockSpec((D_MODEL, blk), lambda i: (0, i))],
        out_specs=pl.BlockSpec((blk, D_PAD), lambda i: (i, 0)),
    )(wt)


def kernel(x, weight):
    b, s = x.shape
    xi = x.astype(jnp.int32)
    idx56 = jnp.concatenate([xi, xi[:, : S_PAD - S_REAL]], axis=1)
    n2 = b * S_PAD
    idx_flat = idx56.reshape(n2)
    table = _tc_prep(weight.T)

    b_chunk = b // N_CHUNKS
    parts = []
    for k in range(N_CHUNKS):
        g = _make_sc_gather_chunk(k, n2)(table, idx_flat)
        g3 = g.reshape(b_chunk, S_PAD, D_PAD)
        parts.append(g3[:, :S_REAL, :D_MODEL])
    return jnp.concatenate(parts, axis=0)


# speedup vs baseline: 1.6190x; 1.4645x over previous
import functools
import math

import jax
import jax.numpy as jnp
from jax import lax
from jax.experimental import pallas as pl
from jax.experimental.pallas import tpu as pltpu
from jax.experimental.pallas import tpu_sc as plsc

D_MODEL = 64
D_PAD = 128
S_REAL = 50
S_PAD = 56
SCALE = math.sqrt(D_MODEL)

NC = 2
NS = 16
NW = NC * NS
N_CHUNKS = 8
CHUNK = 896


def _make_sc_gather_chunk(k, n2):
    rows_per_chunk = n2 // N_CHUNKS
    b_per_w = rows_per_chunk // NW
    assert b_per_w % CHUNK == 0
    n_steps = b_per_w // CHUNK
    mesh = plsc.VectorSubcoreMesh(core_axis_name="c", subcore_axis_name="s")

    @functools.partial(
        pl.kernel,
        mesh=mesh,
        out_type=jax.ShapeDtypeStruct((rows_per_chunk, D_PAD), jnp.float32),
        scratch_types=[
            pltpu.VMEM((CHUNK,), jnp.int32),
            pltpu.VMEM((CHUNK, D_PAD), jnp.float32),
            pltpu.SemaphoreType.DMA,
        ],
    )
    def gather_kernel(table_hbm, idx_hbm, out_hbm, idx_v, rows_v, sem):
        wid = lax.axis_index("s") * NC + lax.axis_index("c")
        base = wid * b_per_w

        @pl.loop(0, n_steps)
        def _(c):
            off = base + c * CHUNK
            pltpu.sync_copy(idx_hbm.at[pl.ds(k * rows_per_chunk + off, CHUNK)], idx_v)
            pltpu.async_copy(table_hbm.at[idx_v], rows_v, sem).wait()
            pltpu.sync_copy(rows_v, out_hbm.at[pl.ds(off, CHUNK)])

    return gather_kernel


def _tc_prep(wt):
    v = wt.shape[1]
    blk = 8192
    grid = (v + blk - 1) // blk

    def body(w_ref, o_ref):
        o_ref[:, :D_MODEL] = w_ref[...].T * SCALE

    return pl.pallas_call(
        body,
        out_shape=jax.ShapeDtypeStruct((v, D_PAD), jnp.float32),
        grid=(grid,),
        in_specs=[pl.BlockSpec((D_MODEL, blk), lambda i: (0, i))],
        out_specs=pl.BlockSpec((blk, D_PAD), lambda i: (i, 0)),
        compiler_params=pltpu.CompilerParams(
            dimension_semantics=("parallel",)
        ),
    )(wt)


def kernel(x, weight):
    b, s = x.shape
    xi = x.astype(jnp.int32)
    idx56 = jnp.concatenate([xi, xi[:, : S_PAD - S_REAL]], axis=1)
    n2 = b * S_PAD
    idx_flat = idx56.reshape(n2)
    table = _tc_prep(weight.T)

    b_chunk = b // N_CHUNKS
    parts = []
    for k in range(N_CHUNKS):
        g = _make_sc_gather_chunk(k, n2)(table, idx_flat)
        g3 = g.reshape(b_chunk, S_PAD, D_PAD)
        parts.append(g3[:, :S_REAL, :D_MODEL])
    return jnp.concatenate(parts, axis=0)
